# baseline (device time: 17257 ns/iter reference)
import jax
import jax.numpy as jnp
from jax import lax
from jax.experimental import pallas as pl
from jax.experimental.pallas import tpu as pltpu

N_DEV = 4
B = 2
SQ = 128
SKV = 128
DH = 64
H_LOC = 4
D_MODEL = 512
QTR = D_MODEL // N_DEV
SCALE = 0.125

SEND_ORDER = (2, 1, 3)


def kernel(x, Wq, K_ext, V_ext, Wo):
    def body(x_ref, wq_ref, k_hbm, v_hbm, wo_ref, out_ref,
             k_scr, v_scr, mine_ref, rs_ref, ag_ref,
             kv_sems, rs_send, rs_recv, ag_send, ag_recv):
        my_pos = lax.axis_index("i")

        kv_copies = []
        for b in range(B):
            for h in range(H_LOC):
                hh = my_pos * H_LOC + h
                idx = b * H_LOC + h
                for src, dst, j in ((k_hbm, k_scr, 0), (v_hbm, v_scr, 1)):
                    c = pltpu.make_async_copy(
                        src.at[b, :, hh, :], dst.at[idx],
                        kv_sems.at[idx, j])
                    c.start()
                    kv_copies.append(c)

        barrier_sem = pltpu.get_barrier_semaphore()
        for d in range(1, N_DEV):
            peer = (my_pos + d) % N_DEV
            pl.semaphore_signal(
                barrier_sem, inc=1,
                device_id=(peer,), device_id_type=pl.DeviceIdType.MESH,
            )

        q_all = jnp.dot(x_ref[...].reshape(B * SQ, x_ref.shape[-1]),
                        wq_ref[...],
                        preferred_element_type=jnp.float32)
        for c in kv_copies:
            c.wait()

        rs_rdmas = [[] for _ in range(B)]
        for b in range(B):
            ctx_parts = []
            for h in range(H_LOC):
                idx = b * H_LOC + h
                qh = q_all[b * SQ:(b + 1) * SQ, h * DH:(h + 1) * DH]
                s = lax.dot_general(
                    qh, k_scr[idx], (((1,), (1,)), ((), ())),
                    preferred_element_type=jnp.float32,
                ) * SCALE
                m = jnp.max(s, axis=-1, keepdims=True)
                e = jnp.exp(s - m)
                w = e / jnp.sum(e, axis=-1, keepdims=True)
                ctx_parts.append(
                    jnp.dot(w, v_scr[idx],
                            preferred_element_type=jnp.float32))
            ctx = jnp.concatenate(ctx_parts, axis=-1)
            partial = jnp.dot(ctx, wo_ref[...],
                              preferred_element_type=jnp.float32)
            for qq in range(N_DEV):
                mine_ref[qq, b] = partial[:, qq * QTR:(qq + 1) * QTR]
            if b == 0:
                pl.semaphore_wait(barrier_sem, N_DEV - 1)
            for d in SEND_ORDER:
                peer = (my_pos + d) % N_DEV
                slot = N_DEV - 1 - d
                rdma = pltpu.make_async_remote_copy(
                    src_ref=mine_ref.at[peer, b],
                    dst_ref=rs_ref.at[slot, b],
                    send_sem=rs_send.at[slot, b],
                    recv_sem=rs_recv.at[slot, b],
                    device_id=(peer,),
                    device_id_type=pl.DeviceIdType.MESH,
                )
                rdma.start()
                rs_rdmas[b].append(rdma)

        ag_rdmas = [[] for _ in range(B)]
        for b in range(B):
            diag, near1, near2 = rs_rdmas[b]
            near1.wait()
            near2.wait()
            red_near = mine_ref[my_pos, b] + rs_ref[0, b] + rs_ref[2, b]
            diag.wait()
            red = red_near + rs_ref[1, b]
            ag_ref[b] = red
            out_ref[b, :, pl.ds(my_pos * QTR, QTR)] = red
            for d in SEND_ORDER:
                peer = (my_pos + d) % N_DEV
                slot = N_DEV - 1 - d
                rdma = pltpu.make_async_remote_copy(
                    src_ref=ag_ref.at[b],
                    dst_ref=out_ref.at[b, :, pl.ds(my_pos * QTR, QTR)],
                    send_sem=ag_send.at[slot, b],
                    recv_sem=ag_recv.at[slot, b],
                    device_id=(peer,),
                    device_id_type=pl.DeviceIdType.MESH,
                )
                rdma.start()
                ag_rdmas[b].append(rdma)

        for b in range(B):
            for rdma in ag_rdmas[b]:
                rdma.wait()

    return pl.pallas_call(
        body,
        out_shape=jax.ShapeDtypeStruct((B, SQ, D_MODEL), jnp.float32),
        in_specs=[
            pl.BlockSpec(memory_space=pltpu.VMEM),
            pl.BlockSpec(memory_space=pltpu.VMEM),
            pl.BlockSpec(memory_space=pltpu.MemorySpace.HBM),
            pl.BlockSpec(memory_space=pltpu.MemorySpace.HBM),
            pl.BlockSpec(memory_space=pltpu.VMEM),
        ],
        out_specs=pl.BlockSpec(memory_space=pltpu.VMEM),
        scratch_shapes=[
            pltpu.VMEM((B * H_LOC, SKV, DH), jnp.float32),
            pltpu.VMEM((B * H_LOC, SKV, DH), jnp.float32),
            pltpu.VMEM((N_DEV, B, SQ, QTR), jnp.float32),
            pltpu.VMEM((N_DEV - 1, B, SQ, QTR), jnp.float32),
            pltpu.VMEM((B, SQ, QTR), jnp.float32),
            pltpu.SemaphoreType.DMA((B * H_LOC, 2)),
            pltpu.SemaphoreType.DMA((N_DEV - 1, B)),
            pltpu.SemaphoreType.DMA((N_DEV - 1, B)),
            pltpu.SemaphoreType.DMA((N_DEV - 1, B)),
            pltpu.SemaphoreType.DMA((N_DEV - 1, B)),
        ],
        compiler_params=pltpu.CompilerParams(collective_id=0),
    )(x, Wq, K_ext, V_ext, Wo)


# device time: 15211 ns/iter; 1.1345x vs baseline; 1.1345x over previous
import jax
import jax.numpy as jnp
from jax import lax
from jax.experimental import pallas as pl
from jax.experimental.pallas import tpu as pltpu

N_DEV = 4
B = 2
SQ = 128
SKV = 128
DH = 64
H_LOC = 4
D_MODEL = 512
QTR = D_MODEL // N_DEV
SCALE = 0.125

SEND_ORDER = (2, 1, 3)


def kernel(x, Wq, K_ext, V_ext, Wo):
    my = lax.axis_index("i")
    Kc = lax.dynamic_slice_in_dim(
        K_ext.reshape(B, SKV, 16 * DH), my * (H_LOC * DH), H_LOC * DH, axis=2)
    Vc = lax.dynamic_slice_in_dim(
        V_ext.reshape(B, SKV, 16 * DH), my * (H_LOC * DH), H_LOC * DH, axis=2)

    def body(x_ref, wq_ref, k_ref, v_ref, wo_ref, out_ref,
             mine_ref, rs_ref, ag_ref,
             rs_send, rs_recv, ag_send, ag_recv):
        my_pos = lax.axis_index("i")

        barrier_sem = pltpu.get_barrier_semaphore()
        for d in range(1, N_DEV):
            peer = (my_pos + d) % N_DEV
            pl.semaphore_signal(
                barrier_sem, inc=1,
                device_id=(peer,), device_id_type=pl.DeviceIdType.MESH,
            )

        q_all = jnp.dot(x_ref[...].reshape(B * SQ, x_ref.shape[-1]),
                        wq_ref[...],
                        preferred_element_type=jnp.float32)
        rs_rdmas = [[] for _ in range(B)]
        for b in range(B):
            ctx_parts = []
            for h in range(H_LOC):
                qh = q_all[b * SQ:(b + 1) * SQ, h * DH:(h + 1) * DH]
                kh = k_ref[b][:, h * DH:(h + 1) * DH]
                s = lax.dot_general(
                    qh, kh, (((1,), (1,)), ((), ())),
                    preferred_element_type=jnp.float32,
                ) * SCALE
                m = jnp.max(s, axis=-1, keepdims=True)
                e = jnp.exp(s - m)
                w = e / jnp.sum(e, axis=-1, keepdims=True)
                ctx_parts.append(
                    jnp.dot(w, v_ref[b][:, h * DH:(h + 1) * DH],
                            preferred_element_type=jnp.float32))
            ctx = jnp.concatenate(ctx_parts, axis=-1)
            partial = jnp.dot(ctx, wo_ref[...],
                              preferred_element_type=jnp.float32)
            for qq in range(N_DEV):
                mine_ref[qq, b] = partial[:, qq * QTR:(qq + 1) * QTR]
            if b == 0:
                pl.semaphore_wait(barrier_sem, N_DEV - 1)
            for d in SEND_ORDER:
                peer = (my_pos + d) % N_DEV
                slot = N_DEV - 1 - d
                rdma = pltpu.make_async_remote_copy(
                    src_ref=mine_ref.at[peer, b],
                    dst_ref=rs_ref.at[slot, b],
                    send_sem=rs_send.at[slot, b],
                    recv_sem=rs_recv.at[slot, b],
                    device_id=(peer,),
                    device_id_type=pl.DeviceIdType.MESH,
                )
                rdma.start()
                rs_rdmas[b].append(rdma)

        ag_rdmas = [[] for _ in range(B)]
        for b in range(B):
            diag, near1, near2 = rs_rdmas[b]
            near1.wait()
            near2.wait()
            red_near = mine_ref[my_pos, b] + rs_ref[0, b] + rs_ref[2, b]
            diag.wait()
            red = red_near + rs_ref[1, b]
            ag_ref[b] = red
            out_ref[b, :, pl.ds(my_pos * QTR, QTR)] = red
            for d in SEND_ORDER:
                peer = (my_pos + d) % N_DEV
                slot = N_DEV - 1 - d
                rdma = pltpu.make_async_remote_copy(
                    src_ref=ag_ref.at[b],
                    dst_ref=out_ref.at[b, :, pl.ds(my_pos * QTR, QTR)],
                    send_sem=ag_send.at[slot, b],
                    recv_sem=ag_recv.at[slot, b],
                    device_id=(peer,),
                    device_id_type=pl.DeviceIdType.MESH,
                )
                rdma.start()
                ag_rdmas[b].append(rdma)

        for b in range(B):
            for rdma in ag_rdmas[b]:
                rdma.wait()

    return pl.pallas_call(
        body,
        out_shape=jax.ShapeDtypeStruct((B, SQ, D_MODEL), jnp.float32),
        in_specs=[pl.BlockSpec(memory_space=pltpu.VMEM)] * 5,
        out_specs=pl.BlockSpec(memory_space=pltpu.VMEM),
        scratch_shapes=[
            pltpu.VMEM((N_DEV, B, SQ, QTR), jnp.float32),
            pltpu.VMEM((N_DEV - 1, B, SQ, QTR), jnp.float32),
            pltpu.VMEM((B, SQ, QTR), jnp.float32),
            pltpu.SemaphoreType.DMA((N_DEV - 1, B)),
            pltpu.SemaphoreType.DMA((N_DEV - 1, B)),
            pltpu.SemaphoreType.DMA((N_DEV - 1, B)),
            pltpu.SemaphoreType.DMA((N_DEV - 1, B)),
        ],
        compiler_params=pltpu.CompilerParams(collective_id=0),
    )(x, Wq, Kc, Vc, Wo)
